# baseline (speedup 1.0000x reference)
import jax
import jax.numpy as jnp
from jax import lax
from jax.experimental import pallas as pl
from jax.experimental.pallas import tpu as pltpu
from jax.experimental.pallas import tpu_sc as plsc

N = 35
E = N * N
W = 48
NS = 16
EPT = 80
EP = NS * EPT
_BN_INV = 1.0 / (1.0 + 0.001) ** 0.5

(_SL_L1W, _SL_L1B, _SL_R1, _SL_B1, _SL_G1, _SL_BB1,
 _SL_L2W, _SL_L2B, _SL_R2,
 _SL_L3W, _SL_L3B, _SL_R3, _SL_B3, _SL_G3, _SL_BB3,
 _SL_X, _SL_SC) = range(17)
_NSLOT = 17
_FW = W * W
_FW2 = W * 16


def _sigmoid(z):
    return 1.0 / (1.0 + jnp.exp(-z))


def _sc_body(prm_hbm, src_hbm, dst_hbm, ea_hbm, msk_hbm,
             out_hbm,
             prm, esrc, edst, eea, emsk,
             acc1, acc2, acc3, x1r, x1m, x2v, x6r, x6m,
             s1f, s2f, s3f, prtb, stR,
             shF1, shF2, shF3, shP1, shP2):
    wid = lax.axis_index("s")
    cid = lax.axis_index("c")
    IOTA = lax.broadcasted_iota(jnp.int32, (16,), 0)
    zero16 = jnp.zeros((16,), jnp.float32)

    base = wid * EPT
    pltpu.sync_copy(prm_hbm, prm)
    pltpu.sync_copy(src_hbm.at[pl.ds(base, EPT)], esrc)
    pltpu.sync_copy(dst_hbm.at[pl.ds(base, EPT)], edst)
    pltpu.sync_copy(ea_hbm.at[pl.ds(base, EPT)], eea)
    pltpu.sync_copy(msk_hbm.at[pl.ds(base, EPT)], emsk)

    def zero_flat(ref, nwords):
        def _z(r, c):
            plsc.store_scatter(ref, [r * 16 + IOTA], zero16)
            return c
        lax.fori_loop(0, nwords // 16, _z, 0)

    zero_flat(acc1, _FW)
    zero_flat(acc2, _FW2)

    def slot(s):
        return [prm[pl.ds(48 * s + 16 * jc, 16)] for jc in range(3)]
    l1W, l1b, root1, bias1, bn1g, bn1b = (slot(s) for s in
                                          (_SL_L1W, _SL_L1B, _SL_R1, _SL_B1, _SL_G1, _SL_BB1))
    l2W, l2b, root2 = slot(_SL_L2W), slot(_SL_L2B), slot(_SL_R2)
    l3W, l3b, root3, bias3, bn3g, bn3b = (slot(s) for s in
                                          (_SL_L3W, _SL_L3B, _SL_R3, _SL_B3, _SL_G3, _SL_BB3))
    def pscal(off):
        return plsc.load_gather(prm, [jnp.full((16,), 48 * _SL_SC + off, jnp.int32)])
    bias2v, bn2gv, bn2bv = pscal(0), pscal(1), pscal(2)
    X_OFF = 48 * _SL_X
    C35 = jnp.full((16,), 35, jnp.int32)

    def reduce_flat(accflat, shP, shF, dstflat, rowlen):
        seg = 3 * rowlen
        nch = seg // 16
        pltpu.sync_copy(accflat, shP.at[pl.ds(wid * W * rowlen, W * rowlen)])
        plsc.subcore_barrier()
        r0 = wid * 3
        for p in range(NS):
            pltpu.sync_copy(shP.at[pl.ds(p * W * rowlen + r0 * rowlen, seg)],
                            prtb.at[pl.ds(p * seg, seg)])
        accv = [zero16] * nch
        for p in range(NS):
            for ch in range(nch):
                accv[ch] = accv[ch] + prtb[pl.ds(p * seg + ch * 16, 16)]
        for ch in range(nch):
            stR[pl.ds(ch * 16, 16)] = accv[ch]
        for ch in range(nch):
            stR[pl.ds(ch * 16, 16)] = accv[ch]
        pltpu.sync_copy(stR.at[pl.ds(0, seg)], shF.at[pl.ds(r0 * rowlen, seg)])
        plsc.subcore_barrier()
        pltpu.sync_copy(shF, dstflat)

    def edge_vectors(e):
        es = jnp.broadcast_to(e, (16,))
        srcs = plsc.load_gather(esrc, [es])
        dsts = plsc.load_gather(edst, [es])
        eas = plsc.load_gather(eea, [es])
        msks = plsc.load_gather(emsk, [es])
        return srcs, dsts, eas, msks

    def _conv1(e, c):
        srcs, dsts, eas, msks = edge_vectors(e)
        xss = plsc.load_gather(prm, [srcs + X_OFF])
        rowb = dsts * W
        for jc in range(3):
            jv = 16 * jc + IOTA
            w = jnp.maximum(eas * l1W[jc] + l1b[jc], 0.0)
            v = msks * xss * w
            if jc == 2:
                v = jnp.where(jv == 35, msks, v)
            plsc.addupdate_scatter(acc1, [rowb + jv], v)
        return c
    lax.fori_loop(0, EPT, _conv1, 0)
    reduce_flat(acc1, shP1, shF1, s1f, W)


    zero_flat(acc3, _FW)
    zero_flat(x1r, _FW)
    zero_flat(x1m, _FW)
    zero_flat(x6r, _FW)
    zero_flat(x6m, _FW)
    x2v[pl.ds(0, 16)] = zero16
    x2v[pl.ds(16, 16)] = zero16
    x2v[pl.ds(32, 16)] = zero16

    def _x1row(n, c):
        rs = jnp.broadcast_to(n, (16,))
        rowb = rs * W
        cnt = plsc.load_gather(s1f, [rowb + C35])
        invc = 1.0 / jnp.maximum(cnt, 1.0)
        xn = plsc.load_gather(prm, [rs + X_OFF])
        for jc in range(3):
            jv = 16 * jc + IOTA
            s1c = plsc.load_gather(s1f, [rowb + jv])
            h = s1c * invc + xn * root1[jc] + bias1[jc]
            v = _sigmoid((_BN_INV * h) * bn1g[jc] + bn1b[jc])
            v = jnp.where(jv < N, v, 0.0)
            plsc.store_scatter(x1r, [rowb + jv], v)
        return c
    lax.fori_loop(0, N, _x1row, 0)

    def _x1sym(n, c):
        rs = jnp.broadcast_to(n, (16,))
        for jc in range(3):
            jv = 16 * jc + IOTA
            a = plsc.load_gather(x1r, [rs * W + jv])
            b = plsc.load_gather(x1r, [jv * W + rs])
            v = 0.5 * (a + b)
            v = jnp.where((jv != rs) & (jv < N), v, 0.0)
            plsc.store_scatter(x1m, [rs * W + jv], v)
        return c
    lax.fori_loop(0, N, _x1sym, 0)


    def _conv2(e, c):
        srcs, dsts, eas, msks = edge_vectors(e)
        p = zero16
        for jc in range(3):
            jv = 16 * jc + IOTA
            xg = plsc.load_gather(x1m, [srcs * W + jv])
            w = jnp.maximum(eas * l2W[jc] + l2b[jc], 0.0)
            p = p + xg * w
        plsc.addupdate_scatter(acc2, [dsts * 16 + IOTA], p * msks)
        return c
    lax.fori_loop(0, EPT, _conv2, 0)
    reduce_flat(acc2, shP2, shF2, s2f, 16)


    def _x2row(n, c):
        rs = jnp.broadcast_to(n, (16,))
        cnt = plsc.load_gather(s1f, [rs * W + C35])
        invc = 1.0 / jnp.maximum(cnt, 1.0)
        dot = zero16
        for jc in range(3):
            jv = 16 * jc + IOTA
            dot = dot + plsc.load_gather(x1m, [rs * W + jv]) * root2[jc]
        dotv = jnp.broadcast_to(jnp.sum(dot), (16,))
        s2n = jnp.broadcast_to(jnp.sum(plsc.load_gather(s2f, [rs * 16 + IOTA])), (16,))
        h = s2n * invc + dotv + bias2v
        v = _sigmoid((_BN_INV * h) * bn2gv + bn2bv)
        plsc.store_scatter(x2v, [rs], v, mask=IOTA == 0)
        return c
    lax.fori_loop(0, N, _x2row, 0)


    def _conv3(e, c):
        srcs, dsts, eas, msks = edge_vectors(e)
        xss = plsc.load_gather(x2v, [srcs])
        rowb = dsts * W
        for jc in range(3):
            jv = 16 * jc + IOTA
            w = jnp.maximum(eas * l3W[jc] + l3b[jc], 0.0)
            plsc.addupdate_scatter(acc3, [rowb + jv], msks * xss * w)
        return c
    lax.fori_loop(0, EPT, _conv3, 0)
    reduce_flat(acc3, shP1, shF3, s3f, W)


    def _x6row(n, c):
        rs = jnp.broadcast_to(n, (16,))
        rowb = rs * W
        cnt = plsc.load_gather(s1f, [rowb + C35])
        invc = 1.0 / jnp.maximum(cnt, 1.0)
        x2n = plsc.load_gather(x2v, [rs])
        for jc in range(3):
            jv = 16 * jc + IOTA
            s3c = plsc.load_gather(s3f, [rowb + jv])
            h = s3c * invc + x2n * root3[jc] + bias3[jc]
            x3 = _sigmoid((_BN_INV * h) * bn3g[jc] + bn3b[jc])
            v = 0.5 * (x3 + plsc.load_gather(x1m, [rowb + jv]))
            v = jnp.where(jv < N, v, 0.0)
            plsc.store_scatter(x6r, [rowb + jv], v)
        return c
    lax.fori_loop(0, N, _x6row, 0)

    def _x6sym(n, c):
        rs = jnp.broadcast_to(n, (16,))
        for jc in range(3):
            jv = 16 * jc + IOTA
            a = plsc.load_gather(x6r, [rs * W + jv])
            b = plsc.load_gather(x6r, [jv * W + rs])
            v = 0.5 * (a + b)
            v = jnp.where((jv != rs) & (jv < N), v, 0.0)
            plsc.store_scatter(x6m, [rs * W + jv], v)
        return c
    lax.fori_loop(0, N, _x6sym, 0)

    @pl.when((wid == 0) & (cid == 0))
    def _():
        pltpu.sync_copy(x6m, out_hbm)


def _sc_x6(prm, src_p, dst_p, ea_p, msk_p):
    mesh = plsc.VectorSubcoreMesh(core_axis_name="c", subcore_axis_name="s")
    f32, i32 = jnp.float32, jnp.int32
    run = pl.kernel(
        _sc_body,
        out_type=jax.ShapeDtypeStruct((_FW,), f32),
        mesh=mesh,
        compiler_params=pltpu.CompilerParams(needs_layout_passes=False),
        scratch_types=[
            pltpu.VMEM((48 * _NSLOT,), f32),
            pltpu.VMEM((EPT,), i32), pltpu.VMEM((EPT,), i32),
            pltpu.VMEM((EPT,), f32), pltpu.VMEM((EPT,), f32),
            pltpu.VMEM((_FW,), f32), pltpu.VMEM((_FW2,), f32),
            pltpu.VMEM((_FW,), f32),
            pltpu.VMEM((_FW,), f32), pltpu.VMEM((_FW,), f32),
            pltpu.VMEM((W,), f32),
            pltpu.VMEM((_FW,), f32), pltpu.VMEM((_FW,), f32),
            pltpu.VMEM((_FW,), f32), pltpu.VMEM((_FW2,), f32),
            pltpu.VMEM((_FW,), f32),
            pltpu.VMEM((NS * 144,), f32), pltpu.VMEM((144,), f32),
            pltpu.VMEM_SHARED((_FW,), f32), pltpu.VMEM_SHARED((_FW2,), f32),
            pltpu.VMEM_SHARED((_FW,), f32),
            pltpu.VMEM_SHARED((NS * _FW,), f32),
            pltpu.VMEM_SHARED((NS * _FW2,), f32),
        ],
    )
    return run(prm, src_p, dst_p, ea_p, msk_p)


def _dot(a, b):
    return jnp.dot(a, b, preferred_element_type=jnp.float32)


def _edge_term_kernel(ea_ref, me1W_ref, me1b_ref, me2W_ref, me2b_ref,
                      me3W_ref, me3b_ref, outW2_ref, outb_ref, o_ref):
    ea = ea_ref[...]
    ef = jax.nn.relu(_dot(ea, me1W_ref[...]) + me1b_ref[...])
    ef = jax.nn.relu(_dot(ef, me2W_ref[...]) + me2b_ref[...])
    ef = _dot(ef, me3W_ref[...]) + me3b_ref[...]
    node_iota = lax.broadcasted_iota(jnp.int32, (N, E), 0)
    e_iota = lax.broadcasted_iota(jnp.int32, (N, E), 1)
    P = (e_iota // N == node_iota).astype(jnp.float32)
    T = (e_iota % N == node_iota).astype(jnp.float32)
    W2e = lax.dot_general(T, outW2_ref[...], (((0,), (0,)), ((), ())),
                          preferred_element_type=jnp.float32)
    o_ref[...] = _dot(P, ef * W2e) + outb_ref[...]


def _final_kernel(x6_ref, outW1_ref, et_ref, o_ref):
    o_ref[...] = _dot(x6_ref[...], outW1_ref[...]) + et_ref[...]


@jax.jit
def _run(x, edge_index, ea, me1_W, me1_b, me2_W, me2_b, me3_W, me3_b,
         out_W, out_b,
         lin1_W, lin1_b, root1, bias1, bn1_g, bn1_b,
         lin2_W, lin2_b, root2, bias2, bn2_g, bn2_b,
         lin3_W, lin3_b, root3, bias3, bn3_g, bn3_b):
    def pad48(v):
        return jnp.pad(v, (0, 48 - v.shape[0]))
    prm = jnp.concatenate([
        pad48(lin1_W[0]), pad48(lin1_b), pad48(root1[0]), pad48(bias1),
        pad48(bn1_g), pad48(bn1_b),
        pad48(lin2_W[0]), pad48(lin2_b), pad48(root2[:, 0]),
        pad48(lin3_W[0]), pad48(lin3_b), pad48(root3[0]), pad48(bias3),
        pad48(bn3_g), pad48(bn3_b),
        pad48(x[:, 0]),
        pad48(jnp.stack([bias2[0], bn2_g[0], bn2_b[0]])),
    ])
    src_p = jnp.pad(edge_index[0], (0, EP - E))
    dst_p = jnp.pad(edge_index[1], (0, EP - E))
    ea_p = jnp.pad(ea[:, 0], (0, EP - E))
    msk_p = jnp.pad(jnp.ones((E,), jnp.float32), (0, EP - E))

    x6 = _sc_x6(prm, src_p, dst_p, ea_p, msk_p).reshape(W, W)[:N, :N]

    edge_term = pl.pallas_call(
        _edge_term_kernel,
        out_shape=jax.ShapeDtypeStruct((N, N), jnp.float32),
    )(ea, me1_W, me1_b.reshape(1, 128), me2_W, me2_b.reshape(1, 256),
      me3_W, me3_b.reshape(1, 1), out_W[N:], out_b.reshape(1, N))

    return pl.pallas_call(
        _final_kernel,
        out_shape=jax.ShapeDtypeStruct((N, N), jnp.float32),
    )(x6, out_W[:N], edge_term)


def kernel(x, edge_index, edge_attr, lin1_W, lin1_b, root1, bias1, bn1_g,
           bn1_b, lin2_W, lin2_b, root2, bias2, bn2_g, bn2_b, lin3_W, lin3_b,
           root3, bias3, bn3_g, bn3_b, me1_W, me1_b, me2_W, me2_b, me3_W,
           me3_b, out_W, out_b):
    return _run(x.astype(jnp.float32), edge_index,
                edge_attr.astype(jnp.float32),
                me1_W, me1_b, me2_W, me2_b, me3_W, me3_b, out_W, out_b,
                lin1_W, lin1_b, root1, bias1, bn1_g, bn1_b,
                lin2_W, lin2_b, root2, bias2, bn2_g, bn2_b,
                lin3_W, lin3_b, root3, bias3, bn3_g, bn3_b)


# speedup vs baseline: 1.1786x; 1.1786x over previous
import jax
import jax.numpy as jnp
from jax import lax
from jax.experimental import pallas as pl
from jax.experimental.pallas import tpu as pltpu
from jax.experimental.pallas import tpu_sc as plsc

N = 35
E = N * N
W = 48
NS = 16
EPT = 80
EP = NS * EPT
_BN_INV = 1.0 / (1.0 + 0.001) ** 0.5

(_SL_L1W, _SL_L1B, _SL_R1, _SL_B1, _SL_G1, _SL_BB1,
 _SL_L2W, _SL_L2B, _SL_R2,
 _SL_L3W, _SL_L3B, _SL_R3, _SL_B3, _SL_G3, _SL_BB3,
 _SL_X, _SL_SC) = range(17)
_NSLOT = 17
_FW = W * W
_FW2 = W * 16


def _sigmoid(z):
    return 1.0 / (1.0 + jnp.exp(-z))


def _sc_body(prm_hbm, src_hbm, dst_hbm, ea_hbm, msk_hbm,
             out_hbm,
             prm, esrc, edst, eea, emsk,
             acc1, acc2, acc3, x1r, x1m, x2v, x6r, x6m,
             s1f, s2f, s3f, prtb, stR, sem,
             shF1, shF2, shF3, shP1, shP2):
    wid = lax.axis_index("s")
    cid = lax.axis_index("c")
    IOTA = lax.broadcasted_iota(jnp.int32, (16,), 0)
    zero16 = jnp.zeros((16,), jnp.float32)

    base = wid * EPT
    handles = [
        pltpu.async_copy(prm_hbm, prm, sem),
        pltpu.async_copy(src_hbm.at[pl.ds(base, EPT)], esrc, sem),
        pltpu.async_copy(dst_hbm.at[pl.ds(base, EPT)], edst, sem),
        pltpu.async_copy(ea_hbm.at[pl.ds(base, EPT)], eea, sem),
        pltpu.async_copy(msk_hbm.at[pl.ds(base, EPT)], emsk, sem),
    ]

    def zero_flat(ref, nwords):
        def _z(r, c):
            plsc.store_scatter(ref, [r * 16 + IOTA], zero16)
            return c
        lax.fori_loop(0, nwords // 16, _z, 0)

    zero_flat(acc1, _FW)
    zero_flat(acc2, _FW2)
    zero_flat(acc3, _FW)
    for h in handles:
        h.wait()

    def slot(s):
        return [prm[pl.ds(48 * s + 16 * jc, 16)] for jc in range(3)]
    l1W, l1b, root1, bias1, bn1g, bn1b = (slot(s) for s in
                                          (_SL_L1W, _SL_L1B, _SL_R1, _SL_B1, _SL_G1, _SL_BB1))
    l2W, l2b, root2 = slot(_SL_L2W), slot(_SL_L2B), slot(_SL_R2)
    l3W, l3b, root3, bias3, bn3g, bn3b = (slot(s) for s in
                                          (_SL_L3W, _SL_L3B, _SL_R3, _SL_B3, _SL_G3, _SL_BB3))
    def pscal(off):
        return plsc.load_gather(prm, [jnp.full((16,), 48 * _SL_SC + off, jnp.int32)])
    bias2v, bn2gv, bn2bv = pscal(0), pscal(1), pscal(2)
    X_OFF = 48 * _SL_X
    C35 = jnp.full((16,), 35, jnp.int32)

    def reduce_flat(accflat, shP, shF, dstflat, rowlen):
        seg = 3 * rowlen
        nch = seg // 16
        pltpu.sync_copy(accflat, shP.at[pl.ds(wid * W * rowlen, W * rowlen)])
        plsc.subcore_barrier()
        r0 = wid * 3
        hs = [pltpu.async_copy(shP.at[pl.ds(p * W * rowlen + r0 * rowlen, seg)],
                               prtb.at[pl.ds(p * seg, seg)], sem)
              for p in range(NS)]
        for h in hs:
            h.wait()
        accv = [zero16] * nch
        for p in range(NS):
            for ch in range(nch):
                accv[ch] = accv[ch] + prtb[pl.ds(p * seg + ch * 16, 16)]
        for ch in range(nch):
            stR[pl.ds(ch * 16, 16)] = accv[ch]
        for ch in range(nch):
            stR[pl.ds(ch * 16, 16)] = accv[ch]
        pltpu.sync_copy(stR.at[pl.ds(0, seg)], shF.at[pl.ds(r0 * rowlen, seg)])
        plsc.subcore_barrier()
        pltpu.sync_copy(shF, dstflat)

    def edge_vectors(e):
        es = jnp.broadcast_to(e, (16,))
        srcs = plsc.load_gather(esrc, [es])
        dsts = plsc.load_gather(edst, [es])
        eas = plsc.load_gather(eea, [es])
        msks = plsc.load_gather(emsk, [es])
        return srcs, dsts, eas, msks

    def _conv1(e, c):
        srcs, dsts, eas, msks = edge_vectors(e)
        xss = plsc.load_gather(prm, [srcs + X_OFF])
        rowb = dsts * W
        for jc in range(3):
            jv = 16 * jc + IOTA
            w = jnp.maximum(eas * l1W[jc] + l1b[jc], 0.0)
            v = msks * xss * w
            if jc == 2:
                v = jnp.where(jv == 35, msks, v)
            plsc.addupdate_scatter(acc1, [rowb + jv], v)
        return c
    lax.fori_loop(0, EPT, _conv1, 0)
    reduce_flat(acc1, shP1, shF1, s1f, W)


    def _x1row(n, c):
        rs = jnp.broadcast_to(n, (16,))
        rowb = rs * W
        cnt = plsc.load_gather(s1f, [rowb + C35])
        invc = 1.0 / jnp.maximum(cnt, 1.0)
        xn = plsc.load_gather(prm, [rs + X_OFF])
        for jc in range(3):
            jv = 16 * jc + IOTA
            s1c = plsc.load_gather(s1f, [rowb + jv])
            h = s1c * invc + xn * root1[jc] + bias1[jc]
            v = _sigmoid((_BN_INV * h) * bn1g[jc] + bn1b[jc])
            v = jnp.where(jv < N, v, 0.0)
            plsc.store_scatter(x1r, [rowb + jv], v)
        return c
    lax.fori_loop(0, N, _x1row, 0)

    def _x1sym(n, c):
        rs = jnp.broadcast_to(n, (16,))
        for jc in range(3):
            jv = 16 * jc + IOTA
            a = plsc.load_gather(x1r, [rs * W + jv])
            b = plsc.load_gather(x1r, [jv * W + rs])
            v = 0.5 * (a + b)
            v = jnp.where((jv != rs) & (jv < N), v, 0.0)
            plsc.store_scatter(x1m, [rs * W + jv], v)
        return c
    lax.fori_loop(0, N, _x1sym, 0)


    def _conv2(e, c):
        srcs, dsts, eas, msks = edge_vectors(e)
        p = zero16
        for jc in range(3):
            jv = 16 * jc + IOTA
            xg = plsc.load_gather(x1m, [srcs * W + jv])
            w = jnp.maximum(eas * l2W[jc] + l2b[jc], 0.0)
            p = p + xg * w
        plsc.addupdate_scatter(acc2, [dsts * 16 + IOTA], p * msks)
        return c
    lax.fori_loop(0, EPT, _conv2, 0)
    reduce_flat(acc2, shP2, shF2, s2f, 16)


    def _x2row(n, c):
        rs = jnp.broadcast_to(n, (16,))
        cnt = plsc.load_gather(s1f, [rs * W + C35])
        invc = 1.0 / jnp.maximum(cnt, 1.0)
        dot = zero16
        for jc in range(3):
            jv = 16 * jc + IOTA
            dot = dot + plsc.load_gather(x1m, [rs * W + jv]) * root2[jc]
        dotv = jnp.broadcast_to(jnp.sum(dot), (16,))
        s2n = jnp.broadcast_to(jnp.sum(plsc.load_gather(s2f, [rs * 16 + IOTA])), (16,))
        h = s2n * invc + dotv + bias2v
        v = _sigmoid((_BN_INV * h) * bn2gv + bn2bv)
        plsc.store_scatter(x2v, [rs], v, mask=IOTA == 0)
        return c
    lax.fori_loop(0, N, _x2row, 0)


    def _conv3(e, c):
        srcs, dsts, eas, msks = edge_vectors(e)
        xss = plsc.load_gather(x2v, [srcs])
        rowb = dsts * W
        for jc in range(3):
            jv = 16 * jc + IOTA
            w = jnp.maximum(eas * l3W[jc] + l3b[jc], 0.0)
            plsc.addupdate_scatter(acc3, [rowb + jv], msks * xss * w)
        return c
    lax.fori_loop(0, EPT, _conv3, 0)
    reduce_flat(acc3, shP1, shF3, s3f, W)


    def _x6row(n, c):
        rs = jnp.broadcast_to(n, (16,))
        rowb = rs * W
        cnt = plsc.load_gather(s1f, [rowb + C35])
        invc = 1.0 / jnp.maximum(cnt, 1.0)
        x2n = plsc.load_gather(x2v, [rs])
        for jc in range(3):
            jv = 16 * jc + IOTA
            s3c = plsc.load_gather(s3f, [rowb + jv])
            h = s3c * invc + x2n * root3[jc] + bias3[jc]
            x3 = _sigmoid((_BN_INV * h) * bn3g[jc] + bn3b[jc])
            v = 0.5 * (x3 + plsc.load_gather(x1m, [rowb + jv]))
            v = jnp.where(jv < N, v, 0.0)
            plsc.store_scatter(x6r, [rowb + jv], v)
        return c
    lax.fori_loop(0, N, _x6row, 0)

    def _x6sym(n, c):
        rs = jnp.broadcast_to(n, (16,))
        for jc in range(3):
            jv = 16 * jc + IOTA
            a = plsc.load_gather(x6r, [rs * W + jv])
            b = plsc.load_gather(x6r, [jv * W + rs])
            v = 0.5 * (a + b)
            v = jnp.where((jv != rs) & (jv < N), v, 0.0)
            plsc.store_scatter(x6m, [rs * W + jv], v)
        return c
    lax.fori_loop(0, N, _x6sym, 0)

    @pl.when((wid == 0) & (cid == 0))
    def _():
        pltpu.sync_copy(x6m, out_hbm)


def _sc_x6(prm, src_p, dst_p, ea_p, msk_p):
    mesh = plsc.VectorSubcoreMesh(core_axis_name="c", subcore_axis_name="s")
    f32, i32 = jnp.float32, jnp.int32
    run = pl.kernel(
        _sc_body,
        out_type=jax.ShapeDtypeStruct((_FW,), f32),
        mesh=mesh,
        compiler_params=pltpu.CompilerParams(needs_layout_passes=False),
        scratch_types=[
            pltpu.VMEM((48 * _NSLOT,), f32),
            pltpu.VMEM((EPT,), i32), pltpu.VMEM((EPT,), i32),
            pltpu.VMEM((EPT,), f32), pltpu.VMEM((EPT,), f32),
            pltpu.VMEM((_FW,), f32), pltpu.VMEM((_FW2,), f32),
            pltpu.VMEM((_FW,), f32),
            pltpu.VMEM((_FW,), f32), pltpu.VMEM((_FW,), f32),
            pltpu.VMEM((W,), f32),
            pltpu.VMEM((_FW,), f32), pltpu.VMEM((_FW,), f32),
            pltpu.VMEM((_FW,), f32), pltpu.VMEM((_FW2,), f32),
            pltpu.VMEM((_FW,), f32),
            pltpu.VMEM((NS * 144,), f32), pltpu.VMEM((144,), f32),
            pltpu.SemaphoreType.DMA,
            pltpu.VMEM_SHARED((_FW,), f32), pltpu.VMEM_SHARED((_FW2,), f32),
            pltpu.VMEM_SHARED((_FW,), f32),
            pltpu.VMEM_SHARED((NS * _FW,), f32),
            pltpu.VMEM_SHARED((NS * _FW2,), f32),
        ],
    )
    return run(prm, src_p, dst_p, ea_p, msk_p)


def _dot(a, b):
    return jnp.dot(a, b, preferred_element_type=jnp.float32)


def _edge_term_kernel(ea_ref, me1W_ref, me1b_ref, me2W_ref, me2b_ref,
                      me3W_ref, me3b_ref, outW2_ref, outb_ref, o_ref):
    ea = ea_ref[...]
    ef = jax.nn.relu(_dot(ea, me1W_ref[...]) + me1b_ref[...])
    ef = jax.nn.relu(_dot(ef, me2W_ref[...]) + me2b_ref[...])
    ef = _dot(ef, me3W_ref[...]) + me3b_ref[...]
    node_iota = lax.broadcasted_iota(jnp.int32, (N, E), 0)
    e_iota = lax.broadcasted_iota(jnp.int32, (N, E), 1)
    P = (e_iota // N == node_iota).astype(jnp.float32)
    T = (e_iota % N == node_iota).astype(jnp.float32)
    W2e = lax.dot_general(T, outW2_ref[...], (((0,), (0,)), ((), ())),
                          preferred_element_type=jnp.float32)
    o_ref[...] = _dot(P, ef * W2e) + outb_ref[...]


def _final_kernel(x6_ref, outW1_ref, et_ref, o_ref):
    o_ref[...] = _dot(x6_ref[...], outW1_ref[...]) + et_ref[...]


@jax.jit
def _run(x, edge_index, ea, me1_W, me1_b, me2_W, me2_b, me3_W, me3_b,
         out_W, out_b,
         lin1_W, lin1_b, root1, bias1, bn1_g, bn1_b,
         lin2_W, lin2_b, root2, bias2, bn2_g, bn2_b,
         lin3_W, lin3_b, root3, bias3, bn3_g, bn3_b):
    def pad48(v):
        return jnp.pad(v, (0, 48 - v.shape[0]))
    prm = jnp.concatenate([
        pad48(lin1_W[0]), pad48(lin1_b), pad48(root1[0]), pad48(bias1),
        pad48(bn1_g), pad48(bn1_b),
        pad48(lin2_W[0]), pad48(lin2_b), pad48(root2[:, 0]),
        pad48(lin3_W[0]), pad48(lin3_b), pad48(root3[0]), pad48(bias3),
        pad48(bn3_g), pad48(bn3_b),
        pad48(x[:, 0]),
        pad48(jnp.stack([bias2[0], bn2_g[0], bn2_b[0]])),
    ])
    src_p = jnp.pad(edge_index[0], (0, EP - E))
    dst_p = jnp.pad(edge_index[1], (0, EP - E))
    ea_p = jnp.pad(ea[:, 0], (0, EP - E))
    msk_p = jnp.pad(jnp.ones((E,), jnp.float32), (0, EP - E))

    x6 = _sc_x6(prm, src_p, dst_p, ea_p, msk_p).reshape(W, W)[:N, :N]

    edge_term = pl.pallas_call(
        _edge_term_kernel,
        out_shape=jax.ShapeDtypeStruct((N, N), jnp.float32),
    )(ea, me1_W, me1_b.reshape(1, 128), me2_W, me2_b.reshape(1, 256),
      me3_W, me3_b.reshape(1, 1), out_W[N:], out_b.reshape(1, N))

    return pl.pallas_call(
        _final_kernel,
        out_shape=jax.ShapeDtypeStruct((N, N), jnp.float32),
    )(x6, out_W[:N], edge_term)


def kernel(x, edge_index, edge_attr, lin1_W, lin1_b, root1, bias1, bn1_g,
           bn1_b, lin2_W, lin2_b, root2, bias2, bn2_g, bn2_b, lin3_W, lin3_b,
           root3, bias3, bn3_g, bn3_b, me1_W, me1_b, me2_W, me2_b, me3_W,
           me3_b, out_W, out_b):
    return _run(x.astype(jnp.float32), edge_index,
                edge_attr.astype(jnp.float32),
                me1_W, me1_b, me2_W, me2_b, me3_W, me3_b, out_W, out_b,
                lin1_W, lin1_b, root1, bias1, bn1_g, bn1_b,
                lin2_W, lin2_b, root2, bias2, bn2_g, bn2_b,
                lin3_W, lin3_b, root3, bias3, bn3_g, bn3_b)
